# baseline (speedup 1.0000x reference)
import functools

import jax
import jax.numpy as jnp
from jax import lax
from jax.experimental import pallas as pl
from jax.experimental.pallas import tpu as pltpu
from jax.experimental.pallas import tpu_sc as plsc

_EMBD = 32
_BATCH = 16384
_HIST = 50

_NC = 2
_NS = 16
_NW = _NC * _NS
_BPW = _BATCH // _NW
_NBUF = 4
_LOOK = 2
_ROWLEN = _HIST * _EMBD

_mesh = plsc.VectorSubcoreMesh(core_axis_name="c", subcore_axis_name="s")


@functools.partial(
    pl.kernel,
    mesh=_mesh,
    out_type=jax.ShapeDtypeStruct((_BATCH * _ROWLEN,), jnp.float32),
    scratch_types=(
        [pltpu.VMEM((_BPW, _HIST), jnp.int32)]
        + [pltpu.VMEM((_HIST, _EMBD), jnp.float32) for _ in range(_NBUF)]
        + [pltpu.VMEM((_ROWLEN,), jnp.float32) for _ in range(_NBUF)]
        + [pltpu.SemaphoreType.DMA for _ in range(2 * _NBUF)]
    ),
    compiler_params=pltpu.CompilerParams(use_tc_tiling_on_sc=False),
)
def _gather_kernel(idx_hbm, table_hbm, out_hbm, idx_v, *bufs_and_sems):
    bufs = bufs_and_sems[:_NBUF]
    obufs = bufs_and_sems[_NBUF : 2 * _NBUF]
    gsem = bufs_and_sems[2 * _NBUF : 3 * _NBUF]
    wsem = bufs_and_sems[3 * _NBUF :]

    wid = lax.axis_index("s") * _NC + lax.axis_index("c")
    base = wid * _BPW
    pltpu.sync_copy(idx_hbm.at[pl.ds(base, _BPW)], idx_v)

    def start_gather(j, slot):
        pltpu.async_copy(table_hbm.at[idx_v.at[j]], bufs[slot], gsem[slot])

    def wait_gather(slot):
        pltpu.make_async_copy(
            table_hbm.at[idx_v.at[0]], bufs[slot], gsem[slot]
        ).wait()

    def relay(slot):
        gbuf, obuf = bufs[slot], obufs[slot]
        for h in range(_HIST):
            obuf[pl.ds(h * _EMBD, 16)] = gbuf[h, pl.ds(0, 16)]
            obuf[pl.ds(h * _EMBD + 16, 16)] = gbuf[h, pl.ds(16, 16)]

    def start_write(j, slot):
        pltpu.async_copy(
            obufs[slot],
            out_hbm.at[pl.ds((base + j) * _ROWLEN, _ROWLEN)],
            wsem[slot],
        )

    def wait_write(slot):
        pltpu.make_async_copy(
            obufs[slot], out_hbm.at[pl.ds(0, _ROWLEN)], wsem[slot]
        ).wait()

    for j in range(_LOOK):
        start_gather(j, j)

    for j in range(_LOOK):
        start_gather(j + _LOOK, j + _LOOK)
        wait_gather(j)
        relay(j)
        start_write(j, j)

    @pl.loop(_LOOK, _BPW - _LOOK, step=_NBUF)
    def _steady(g):
        for b in range(_NBUF):
            j = g + b
            s_ahead = (_LOOK + b + _LOOK) % _NBUF
            wait_write(s_ahead)
            start_gather(j + _LOOK, s_ahead)
            slot = (_LOOK + b) % _NBUF
            wait_gather(slot)
            relay(slot)
            start_write(j, slot)

    for t in range(_LOOK):
        slot = (_BPW - _LOOK + t) % _NBUF
        wait_gather(slot)
        relay(slot)
        start_write(_BPW - _LOOK + t, slot)

    for b in range(_NBUF):
        wait_write(b)


_FB = 32
_FROWS = _FB * _ROWLEN // 128


def _format_body(in_ref, out_ref):
    x = in_ref[...]
    parts = [x[:, 32 * k : 32 * (k + 1)] for k in range(4)]
    y = jnp.stack(parts, axis=1)
    out_ref[...] = y.reshape(_FB, _HIST, _EMBD)


_format = pl.pallas_call(
    _format_body,
    grid=(_BATCH // _FB,),
    in_specs=[pl.BlockSpec((_FROWS, 128), lambda i: (i, 0))],
    out_specs=pl.BlockSpec((_FB, _HIST, _EMBD), lambda i: (i, 0, 0)),
    out_shape=jax.ShapeDtypeStruct((_BATCH, _HIST, _EMBD), jnp.float32),
)


def kernel(inputs, context_weight):
    flat = _gather_kernel(inputs.astype(jnp.int32), context_weight)
    return _format(flat.reshape(_BATCH * _ROWLEN // 128, 128))


# speedup vs baseline: 1.5645x; 1.5645x over previous
import functools

import jax
import jax.numpy as jnp
from jax import lax
from jax.experimental import pallas as pl
from jax.experimental.pallas import tpu as pltpu
from jax.experimental.pallas import tpu_sc as plsc

_EMBD = 32
_BATCH = 16384
_HIST = 50

_NC = 2
_NS = 16
_NW = _NC * _NS
_BPW = _BATCH // _NW
_NBUF = 8
_LOOK = 4

_mesh = plsc.VectorSubcoreMesh(core_axis_name="c", subcore_axis_name="s")


@functools.partial(
    pl.kernel,
    mesh=_mesh,
    out_type=jax.ShapeDtypeStruct((_BATCH, _HIST, _EMBD), jnp.float32),
    scratch_types=(
        [pltpu.VMEM((_BPW, _HIST), jnp.int32)]
        + [pltpu.VMEM((_HIST, _EMBD), jnp.float32) for _ in range(_NBUF)]
        + [pltpu.SemaphoreType.DMA for _ in range(2 * _NBUF)]
    ),
    compiler_params=pltpu.CompilerParams(use_tc_tiling_on_sc=False),
)
def _gather_kernel(idx_hbm, table_hbm, out_hbm, idx_v, *bufs_and_sems):
    bufs = bufs_and_sems[:_NBUF]
    gsem = bufs_and_sems[_NBUF : 2 * _NBUF]
    wsem = bufs_and_sems[2 * _NBUF :]

    wid = lax.axis_index("s") * _NC + lax.axis_index("c")
    base = wid * _BPW
    pltpu.sync_copy(idx_hbm.at[pl.ds(base, _BPW)], idx_v)

    def start_gather(j, slot):
        pltpu.async_copy(table_hbm.at[idx_v.at[j]], bufs[slot], gsem[slot])

    def wait_gather(slot):
        pltpu.make_async_copy(
            table_hbm.at[idx_v.at[0]], bufs[slot], gsem[slot]
        ).wait()

    def start_write(j, slot):
        pltpu.async_copy(bufs[slot], out_hbm.at[base + j], wsem[slot])

    def wait_write(slot):
        pltpu.make_async_copy(bufs[slot], out_hbm.at[0], wsem[slot]).wait()

    for j in range(_LOOK):
        start_gather(j, j)

    for j in range(_LOOK):
        start_gather(j + _LOOK, j + _LOOK)
        wait_gather(j)
        start_write(j, j)

    @pl.loop(_LOOK, _BPW - _LOOK, step=_NBUF)
    def _steady(g):
        for b in range(_NBUF):
            j = g + b
            s_ahead = (_LOOK + b + _LOOK) % _NBUF
            wait_write(s_ahead)
            start_gather(j + _LOOK, s_ahead)
            slot = (_LOOK + b) % _NBUF
            wait_gather(slot)
            start_write(j, slot)

    for t in range(_LOOK):
        slot = (_BPW - _LOOK + t) % _NBUF
        wait_gather(slot)
        start_write(_BPW - _LOOK + t, slot)

    for b in range(_NBUF):
        wait_write(b)


def kernel(inputs, context_weight):
    return _gather_kernel(inputs.astype(jnp.int32), context_weight)
